# baseline (speedup 1.0000x reference)
import functools

import jax
import jax.numpy as jnp
from jax import lax
from jax.experimental import pallas as pl
from jax.experimental.pallas import tpu as pltpu
from jax.experimental.pallas import tpu_sc as plsc

BATCH = 16384
FACTORS = 64
R = 1000000

_info = plsc.get_sparse_core_info()
NC = _info.num_cores
NS = _info.num_subcores
NW = NC * NS

TCOLS = 7813
COLS_PER_TILE = 245
PIECE_W = 640
PIECES = 49
LIST_CAP = 1024
BUCKET_CAP = 64
TAIL_LO = 999936
DUMMY_ROW = BATCH
STAGE_ROWS = 16896


def _stage_a(user_hbm, item_hbm, euT_hbm, eiT_hbm, auxu_hbm, auxi_hbm,
             stu_hbm, sti_hbm,
             idx_c, lb, lr, ib, ir, piece_v, bb, br, bidx, blk, aux_v):
    wid = lax.axis_index("s") * NC + lax.axis_index("c")
    col_lo = wid * COLS_PER_TILE
    col_hi = jnp.minimum(col_lo + COLS_PER_TILE, TCOLS - 1)
    r_lo = col_lo * 128
    r_hi = col_hi * 128
    is_last = wid == (NW - 1)
    r_hi_eff = jnp.where(is_last, R, r_hi)

    lanes = lax.iota(jnp.int32, 16)
    minus1 = jnp.full((16,), -1, jnp.int32)
    dummy16 = jnp.full((16,), DUMMY_ROW, jnp.int32)

    def prefill(i, _):
        sl = pl.ds(i * 16, 16)
        lb[sl] = dummy16
        lr[sl] = minus1
        ib[sl] = dummy16
        ir[sl] = minus1
        return 0
    lax.fori_loop(0, LIST_CAP // 16, prefill, 0)

    def build_list(src_hbm, dst_b, dst_r):
        cnt = 0
        for c in range(BATCH // 2048):
            pltpu.sync_copy(src_hbm.at[pl.ds(c * 2048, 2048)], idx_c)

            def grp(i, cnt):
                r16 = idx_c[pl.ds(i * 16, 16)]
                m = (r16 >= r_lo) & (r16 < r_hi_eff)
                b16 = lanes + (c * 2048 + i * 16)
                plsc.store_compressed(dst_b.at[pl.ds(cnt, 16)], b16, mask=m)
                plsc.store_compressed(dst_r.at[pl.ds(cnt, 16)], r16, mask=m)
                pc = plsc.all_reduce_population_count(m)
                return cnt + pc[0]

            cnt = lax.fori_loop(0, 2048 // 16, grp, cnt)
        return cnt

    ucnt = build_list(user_hbm, lb, lr)
    icnt = build_list(item_hbm, ib, ir)

    def extract(panel, b16, r16, base, st_hbm):
        m = r16 >= 0
        rl = jnp.maximum(r16 - base, 0)

        def fx(f, _):
            fb = jnp.full((16,), f, jnp.int32)
            v = plsc.load_gather(panel, [fb, rl])
            plsc.store_scatter(blk, [lanes, fb], v, mask=m)
            return 0
        lax.fori_loop(0, FACTORS, fx, 0, unroll=8)
        bidx[...] = jnp.where(m, b16, dummy16)
        pltpu.sync_copy(blk, st_hbm.at[bidx])

    def scan_table(src_hbm, st_hbm, list_b, list_r, cnt1):
        def piece(p, _):
            m_lo = r_lo + p * PIECE_W
            m_hi = jnp.minimum(m_lo + PIECE_W, r_hi)

            @pl.when(m_lo < r_hi)
            def _():
                win = pl.multiple_of(jnp.minimum(m_lo, r_hi - PIECE_W), 128)
                pltpu.sync_copy(
                    src_hbm.at[pl.ds(0, FACTORS), pl.ds(win, PIECE_W)],
                    piece_v)
                bb[pl.ds(0, 16)] = dummy16
                br[pl.ds(0, 16)] = minus1
                for q in range(1, BUCKET_CAP // 16):
                    sl = pl.ds(q * 16, 16)
                    bb[sl] = dummy16
                    br[sl] = minus1

                def bscan(g, cb):
                    def hit(cb):
                        sl = pl.ds(g * 16, 16)
                        r16 = list_r[sl]
                        b16 = list_b[sl]
                        m = (r16 >= m_lo) & (r16 < m_hi)
                        plsc.store_compressed(bb.at[pl.ds(cb, 16)], b16, mask=m)
                        plsc.store_compressed(br.at[pl.ds(cb, 16)], r16, mask=m)
                        pc = plsc.all_reduce_population_count(m)
                        return cb + pc[0]
                    return lax.cond(g * 16 < cnt1, hit, lambda cb: cb, cb)

                cb = lax.fori_loop(0, LIST_CAP // 16, bscan, 0)

                for g in range(BUCKET_CAP // 16):
                    @pl.when(cb > g * 16)
                    def _():
                        sl = pl.ds(g * 16, 16)
                        extract(piece_v, bb[sl], br[sl], win, st_hbm)
            return 0
        lax.fori_loop(0, PIECES, piece, 0)

    scan_table(euT_hbm, stu_hbm, lb, lr, ucnt)
    scan_table(eiT_hbm, sti_hbm, ib, ir, icnt)

    @pl.when(is_last)
    def _():
        for src_hbm, st_hbm, list_b, list_r, cnt1 in (
            (auxu_hbm, stu_hbm, lb, lr, ucnt),
            (auxi_hbm, sti_hbm, ib, ir, icnt),
        ):
            pltpu.sync_copy(src_hbm, aux_v)
            bb[pl.ds(0, 16)] = dummy16
            br[pl.ds(0, 16)] = minus1

            def tscan(g, cb):
                def hit(cb):
                    sl = pl.ds(g * 16, 16)
                    r16 = list_r[sl]
                    m = r16 >= TAIL_LO
                    plsc.store_compressed(bb.at[pl.ds(cb, 16)], list_b[sl], mask=m)
                    plsc.store_compressed(br.at[pl.ds(cb, 16)], r16, mask=m)
                    pc = plsc.all_reduce_population_count(m)
                    return cb + pc[0]
                return lax.cond(g * 16 < cnt1, hit, lambda cb: cb, cb)

            cb = lax.fori_loop(0, LIST_CAP // 16, tscan, 0)

            @pl.when(cb > 0)
            def _():
                sl = pl.ds(0, 16)
                extract(aux_v, bb[sl], br[sl], TAIL_LO, st_hbm)


def _stage_b(u_ref, i_ref, o_ref):
    u = u_ref[:, :FACTORS]
    v = i_ref[:, :FACTORS]
    o_ref[...] = jnp.sum(u * v, axis=1)


@jax.jit
def _run(user, item, euT, eiT, aux_u, aux_i):
    mesh = plsc.VectorSubcoreMesh(core_axis_name="c", subcore_axis_name="s")
    fa = functools.partial(
        pl.kernel,
        mesh=mesh,
        compiler_params=pltpu.CompilerParams(
            needs_layout_passes=False, use_tc_tiling_on_sc=True),
        out_type=(
            jax.ShapeDtypeStruct((STAGE_ROWS, 128), jnp.float32),
            jax.ShapeDtypeStruct((STAGE_ROWS, 128), jnp.float32),
        ),
        scratch_types=[
            pltpu.VMEM((2048,), jnp.int32),
            pltpu.VMEM((LIST_CAP,), jnp.int32),
            pltpu.VMEM((LIST_CAP,), jnp.int32),
            pltpu.VMEM((LIST_CAP,), jnp.int32),
            pltpu.VMEM((LIST_CAP,), jnp.int32),
            pltpu.VMEM((FACTORS, PIECE_W), jnp.float32),
            pltpu.VMEM((BUCKET_CAP,), jnp.int32),
            pltpu.VMEM((BUCKET_CAP,), jnp.int32),
            pltpu.VMEM((16,), jnp.int32),
            pltpu.VMEM((16, 128), jnp.float32),
            pltpu.VMEM((FACTORS, 128), jnp.float32),
        ],
    )(_stage_a)
    st_u, st_i = fa(user, item, euT, eiT, aux_u, aux_i)

    out = pl.pallas_call(
        _stage_b,
        grid=(STAGE_ROWS // 512,),
        in_specs=[
            pl.BlockSpec((512, 128), lambda i: (i, 0)),
            pl.BlockSpec((512, 128), lambda i: (i, 0)),
        ],
        out_specs=pl.BlockSpec((512,), lambda i: (i,)),
        out_shape=jax.ShapeDtypeStruct((STAGE_ROWS,), jnp.float32),
    )(st_u, st_i)
    return out[:BATCH]


def kernel(user, item, embed_user, embed_item):
    u = user.astype(jnp.int32)
    it = item.astype(jnp.int32)
    aux_u = jnp.pad(embed_user[TAIL_LO:, :].T, ((0, 0), (0, 64)))
    aux_i = jnp.pad(embed_item[TAIL_LO:, :].T, ((0, 0), (0, 64)))
    return _run(u, it, embed_user.T, embed_item.T, aux_u, aux_i)


# speedup vs baseline: 1.4273x; 1.4273x over previous
import functools

import jax
import jax.numpy as jnp
from jax import lax
from jax.experimental import pallas as pl
from jax.experimental.pallas import tpu as pltpu
from jax.experimental.pallas import tpu_sc as plsc

BATCH = 16384
FACTORS = 64
R = 1000000

_info = plsc.get_sparse_core_info()
NC = _info.num_cores
NS = _info.num_subcores
NW = NC * NS

TCOLS = 7813
COLS_PER_TILE = 245
PIECE_W = 640
PIECES = 49
LIST_CAP = 1024
BUCKET_CAP = 64
TAIL_LO = 999936
DUMMY_ROW = BATCH
STAGE_ROWS = 16896


def _stage_a(user_hbm, item_hbm, euT_hbm, eiT_hbm, auxu_hbm, auxi_hbm,
             stu_hbm, sti_hbm,
             idx_c, lb, lr, ib, ir, piece_u, piece_i, bb, br, bidx, blk,
             aux_v, sem_u, sem_i):
    wid = lax.axis_index("s") * NC + lax.axis_index("c")
    col_lo = wid * COLS_PER_TILE
    col_hi = jnp.minimum(col_lo + COLS_PER_TILE, TCOLS - 1)
    r_lo = col_lo * 128
    r_hi = col_hi * 128
    is_last = wid == (NW - 1)
    r_hi_eff = jnp.where(is_last, R, r_hi)

    lanes = lax.iota(jnp.int32, 16)
    minus1 = jnp.full((16,), -1, jnp.int32)
    dummy16 = jnp.full((16,), DUMMY_ROW, jnp.int32)

    def prefill(i, _):
        sl = pl.ds(i * 16, 16)
        lb[sl] = dummy16
        lr[sl] = minus1
        ib[sl] = dummy16
        ir[sl] = minus1
        return 0
    lax.fori_loop(0, LIST_CAP // 16, prefill, 0)

    def build_list(src_hbm, dst_b, dst_r):
        cnt = 0
        for c in range(BATCH // 2048):
            pltpu.sync_copy(src_hbm.at[pl.ds(c * 2048, 2048)], idx_c)

            def grp(i, cnt):
                r16 = idx_c[pl.ds(i * 16, 16)]
                m = (r16 >= r_lo) & (r16 < r_hi_eff)
                b16 = lanes + (c * 2048 + i * 16)
                plsc.store_compressed(dst_b.at[pl.ds(cnt, 16)], b16, mask=m)
                plsc.store_compressed(dst_r.at[pl.ds(cnt, 16)], r16, mask=m)
                pc = plsc.all_reduce_population_count(m)
                return cnt + pc[0]

            cnt = lax.fori_loop(0, 2048 // 16, grp, cnt)
        return cnt

    ucnt = build_list(user_hbm, lb, lr)
    icnt = build_list(item_hbm, ib, ir)

    def extract(panel, b16, r16, base, st_hbm):
        m = r16 >= 0
        rl = jnp.maximum(r16 - base, 0)

        def fx(f, _):
            fb = jnp.full((16,), f, jnp.int32)
            v = plsc.load_gather(panel, [fb, rl])
            plsc.store_scatter(blk, [lanes, fb], v, mask=m)
            return 0
        lax.fori_loop(0, FACTORS, fx, 0, unroll=8)
        bidx[...] = jnp.where(m, b16, dummy16)
        pltpu.sync_copy(blk, st_hbm.at[bidx])

    def win_of(p):
        m_lo = r_lo + p * PIECE_W
        return pl.multiple_of(jnp.minimum(m_lo, r_hi - PIECE_W), 128)

    def fire(src_hbm, p, dst, sem):
        pltpu.async_copy(
            src_hbm.at[pl.ds(0, FACTORS), pl.ds(win_of(p), PIECE_W)],
            dst, sem)

    def drain(dst, sem):
        pltpu.make_async_copy(
            euT_hbm.at[pl.ds(0, FACTORS), pl.ds(0, PIECE_W)], dst, sem
        ).wait()

    def process(panel, list_b, list_r, cnt1, st_hbm, p):
        m_lo = r_lo + p * PIECE_W
        m_hi = jnp.minimum(m_lo + PIECE_W, r_hi)
        win = win_of(p)
        bb[pl.ds(0, 16)] = dummy16
        br[pl.ds(0, 16)] = minus1
        for q in range(1, BUCKET_CAP // 16):
            sl = pl.ds(q * 16, 16)
            bb[sl] = dummy16
            br[sl] = minus1

        def bscan(g, cb):
            def hit(cb):
                sl = pl.ds(g * 16, 16)
                r16 = list_r[sl]
                b16 = list_b[sl]
                m = (r16 >= m_lo) & (r16 < m_hi)
                plsc.store_compressed(bb.at[pl.ds(cb, 16)], b16, mask=m)
                plsc.store_compressed(br.at[pl.ds(cb, 16)], r16, mask=m)
                pc = plsc.all_reduce_population_count(m)
                return cb + pc[0]
            return lax.cond(g * 16 < cnt1, hit, lambda cb: cb, cb)

        cb = lax.fori_loop(0, LIST_CAP // 16, bscan, 0)

        for g in range(BUCKET_CAP // 16):
            @pl.when(cb > g * 16)
            def _():
                sl = pl.ds(g * 16, 16)
                extract(panel, bb[sl], br[sl], win, st_hbm)

    fire(euT_hbm, 0, piece_u, sem_u)

    def piece(p, _):
        drain(piece_u, sem_u)
        fire(eiT_hbm, p, piece_i, sem_i)
        process(piece_u, lb, lr, ucnt, stu_hbm, p)
        drain(piece_i, sem_i)

        @pl.when(p < PIECES - 1)
        def _():
            fire(euT_hbm, p + 1, piece_u, sem_u)
        process(piece_i, ib, ir, icnt, sti_hbm, p)
        return 0
    lax.fori_loop(0, PIECES, piece, 0)

    @pl.when(is_last)
    def _():
        for src_hbm, st_hbm, list_b, list_r, cnt1 in (
            (auxu_hbm, stu_hbm, lb, lr, ucnt),
            (auxi_hbm, sti_hbm, ib, ir, icnt),
        ):
            pltpu.sync_copy(src_hbm, aux_v)
            bb[pl.ds(0, 16)] = dummy16
            br[pl.ds(0, 16)] = minus1

            def tscan(g, cb):
                def hit(cb):
                    sl = pl.ds(g * 16, 16)
                    r16 = list_r[sl]
                    m = r16 >= TAIL_LO
                    plsc.store_compressed(bb.at[pl.ds(cb, 16)], list_b[sl], mask=m)
                    plsc.store_compressed(br.at[pl.ds(cb, 16)], r16, mask=m)
                    pc = plsc.all_reduce_population_count(m)
                    return cb + pc[0]
                return lax.cond(g * 16 < cnt1, hit, lambda cb: cb, cb)

            cb = lax.fori_loop(0, LIST_CAP // 16, tscan, 0)

            @pl.when(cb > 0)
            def _():
                sl = pl.ds(0, 16)
                extract(aux_v, bb[sl], br[sl], TAIL_LO, st_hbm)


def _stage_b(u_ref, i_ref, o_ref):
    u = u_ref[:, :FACTORS]
    v = i_ref[:, :FACTORS]
    o_ref[...] = jnp.sum(u * v, axis=1)


@jax.jit
def _run(user, item, euT, eiT, aux_u, aux_i):
    mesh = plsc.VectorSubcoreMesh(core_axis_name="c", subcore_axis_name="s")
    fa = functools.partial(
        pl.kernel,
        mesh=mesh,
        compiler_params=pltpu.CompilerParams(
            needs_layout_passes=False, use_tc_tiling_on_sc=True),
        out_type=(
            jax.ShapeDtypeStruct((STAGE_ROWS, 128), jnp.float32),
            jax.ShapeDtypeStruct((STAGE_ROWS, 128), jnp.float32),
        ),
        scratch_types=[
            pltpu.VMEM((2048,), jnp.int32),
            pltpu.VMEM((LIST_CAP,), jnp.int32),
            pltpu.VMEM((LIST_CAP,), jnp.int32),
            pltpu.VMEM((LIST_CAP,), jnp.int32),
            pltpu.VMEM((LIST_CAP,), jnp.int32),
            pltpu.VMEM((FACTORS, PIECE_W), jnp.float32),
            pltpu.VMEM((FACTORS, PIECE_W), jnp.float32),
            pltpu.VMEM((BUCKET_CAP,), jnp.int32),
            pltpu.VMEM((BUCKET_CAP,), jnp.int32),
            pltpu.VMEM((16,), jnp.int32),
            pltpu.VMEM((16, 128), jnp.float32),
            pltpu.VMEM((FACTORS, 128), jnp.float32),
            pltpu.SemaphoreType.DMA,
            pltpu.SemaphoreType.DMA,
        ],
    )(_stage_a)
    st_u, st_i = fa(user, item, euT, eiT, aux_u, aux_i)

    out = pl.pallas_call(
        _stage_b,
        grid=(STAGE_ROWS // 512,),
        in_specs=[
            pl.BlockSpec((512, 128), lambda i: (i, 0)),
            pl.BlockSpec((512, 128), lambda i: (i, 0)),
        ],
        out_specs=pl.BlockSpec((512,), lambda i: (i,)),
        out_shape=jax.ShapeDtypeStruct((STAGE_ROWS,), jnp.float32),
    )(st_u, st_i)
    return out[:BATCH]


def kernel(user, item, embed_user, embed_item):
    u = user.astype(jnp.int32)
    it = item.astype(jnp.int32)
    aux_u = jnp.pad(embed_user[TAIL_LO:, :].T, ((0, 0), (0, 64)))
    aux_i = jnp.pad(embed_item[TAIL_LO:, :].T, ((0, 0), (0, 64)))
    return _run(u, it, embed_user.T, embed_item.T, aux_u, aux_i)
